# baseline (speedup 1.0000x reference)
import functools
import math

import jax
import jax.numpy as jnp
from jax import lax
from jax.experimental import pallas as pl
from jax.experimental.pallas import tpu as pltpu
from jax.experimental.pallas import tpu_sc as plsc

_NUM_DOSES = 100
_B = 16384
_D = 64
_V = 1000000
_NC = 2
_NS = 16
_NW = _NC * _NS
_L = 16
_NBLK = (_V + 127) // 128
_BPB = _NBLK // _NW
_CAP = 768
_NSUB = 16
_SB = 16
_BKCAP = 80
_CHF = 2048
_SENT = 0x7FFFFFFF
_PSENT = _B


def _sc_scan_select(dev, dose, table_t):
    mesh = plsc.VectorSubcoreMesh(core_axis_name="c", subcore_axis_name="s")

    @functools.partial(
        pl.kernel,
        mesh=mesh,
        out_type=(
            jax.ShapeDtypeStruct((_NW * _NSUB * _BKCAP, _D), jnp.float32),
            jax.ShapeDtypeStruct((_NW * _NSUB * _BKCAP,), jnp.int32),
        ),
        scratch_types=[
            pltpu.VMEM((_CHF,), jnp.int32),
            pltpu.VMEM((_CHF,), jnp.int32),
            pltpu.VMEM((_CAP + _L,), jnp.int32),
            pltpu.VMEM((_CAP + _L,), jnp.int32),
            pltpu.VMEM((_NSUB, _BKCAP + _L), jnp.int32),
            pltpu.VMEM((_NSUB, _BKCAP + _L), jnp.int32),
            pltpu.VMEM((_BKCAP + _L,), jnp.int32),
            pltpu.VMEM((_BKCAP + _L,), jnp.int32),
            pltpu.VMEM((64, 128), jnp.float32),
            pltpu.VMEM((_BKCAP, _D), jnp.float32),
        ],
        compiler_params=pltpu.CompilerParams(
            use_tc_tiling_on_sc=True, needs_layout_passes=False),
    )
    def k(dev_hbm, dose_hbm, table_hbm, staged_hbm, pos_hbm,
          dv, sv, oidx, opos, bidx, bpos, mrow, ment, buf, rows_s):
        wid = lax.axis_index("s") * _NC + lax.axis_index("c")
        nb = _BPB + ((wid + 1) // _NW) * (_NBLK - _NW * _BPB)
        lo_b = wid * _BPB
        iota = lax.iota(jnp.int32, _L)
        sentv = jnp.full((_L,), _SENT, jnp.int32)
        psentv = jnp.full((_L,), _PSENT, jnp.int32)

        def initb(v, c):
            oidx[pl.ds(v * _L, _L)] = sentv
            return c
        lax.fori_loop(0, (_CAP + _L) // _L, initb, 0)
        for s in range(_NSUB):
            def initp(v, c, s=s):
                bpos[s, pl.ds(v * _L, _L)] = psentv
                bidx[s, pl.ds(v * _L, _L)] = sentv
                return c
            lax.fori_loop(0, (_BKCAP + _L) // _L, initp, 0)

        def chunk(ci, cnt):
            pltpu.sync_copy(dev_hbm.at[pl.ds(ci * _CHF, _CHF)], dv)
            pltpu.sync_copy(dose_hbm.at[pl.ds(ci * _CHF, _CHF)], sv)

            def vec(vi, cnt):
                sl = pl.ds(vi * _L, _L)
                c = dv[sl] * _NUM_DOSES + sv[sl]
                b = lax.shift_right_logical(c, 7)
                m = jnp.logical_and(b >= lo_b, b < lo_b + nb)
                key = jnp.where(m, c, jnp.int32(_SENT))
                p = iota + (ci * _CHF + vi * _L)
                sk, sp = plsc.sort_key_val(key, p)
                oidx[pl.ds(cnt, _L)] = sk
                opos[pl.ds(cnt, _L)] = sp
                return lax.min(cnt + jnp.sum(m.astype(jnp.int32)), _CAP)

            return lax.fori_loop(0, _CHF // _L, vec, cnt)

        cnt = lax.fori_loop(0, _B // _CHF, chunk, 0)
        nvec = lax.shift_right_logical(cnt + _L - 1, 4)

        bcs = []
        for s in range(_NSUB):
            def bucket(v, bc, s=s):
                ob = oidx[pl.ds(v * _L, _L)]
                op = opos[pl.ds(v * _L, _L)]
                sub = lax.shift_right_logical(
                    lax.shift_right_logical(ob, 7) - lo_b, 4)
                m = sub == jnp.full((_L,), s, jnp.int32)
                key = jnp.where(m, ob, jnp.int32(_SENT))
                sk, sp = plsc.sort_key_val(key, jnp.where(m, op, psentv))
                bidx[s, pl.ds(bc, _L)] = sk
                bpos[s, pl.ds(bc, _L)] = sp
                return lax.min(bc + jnp.sum(m.astype(jnp.int32)), _BKCAP)

            bcs.append(lax.fori_loop(0, nvec, bucket, 0))

        for s in range(_NSUB):
            nv_s = lax.shift_right_logical(bcs[s] + _L - 1, 4)

            def block(blk, carry, s=s, nv_s=nv_s):
                off = pl.multiple_of(blk * 128, 128)
                pltpu.sync_copy(table_hbm.at[:, pl.ds(off, 128)], buf)
                blkv = jnp.full((_L,), blk, jnp.int32)

                def scan(v, mcnt):
                    ob = bidx[s, pl.ds(v * _L, _L)]
                    m = lax.shift_right_logical(ob, 7) == blkv
                    key = jnp.where(m, lax.bitwise_and(ob, 127),
                                    jnp.int32(_SENT))
                    sk, se = plsc.sort_key_val(key, iota + v * _L)
                    mrow[pl.ds(mcnt, _L)] = sk
                    ment[pl.ds(mcnt, _L)] = se
                    return mcnt + jnp.sum(m.astype(jnp.int32))

                mcnt = lax.fori_loop(0, nv_s, scan, 0)

                def sel(e2, carry2):
                    r = mrow[pl.ds(e2, _L)][0]
                    e = ment[pl.ds(e2, _L)][0]
                    rsp = jnp.full((_L,), r, jnp.int32)
                    for g in range(_D // _L):
                        col = plsc.load_gather(buf, [iota + g * _L, rsp])
                        rows_s[e, pl.ds(g * _L, _L)] = col
                    return carry2

                lax.fori_loop(0, mcnt, sel, 0)
                return carry

            lo_s = lo_b + s * _SB
            hi_s = lo_b + lax.min((s + 1) * _SB, nb)
            lax.fori_loop(lo_s, hi_s, block, 0)

            base = wid * _NSUB * _BKCAP + s * _BKCAP
            pltpu.sync_copy(rows_s, staged_hbm.at[pl.ds(base, _BKCAP)])
            pltpu.sync_copy(bpos.at[s, pl.ds(0, _BKCAP)],
                            pos_hbm.at[pl.ds(base, _BKCAP)])

    return k(dev, dose, table_t)


def _sc_scatter(staged, pos):
    mesh = plsc.VectorSubcoreMesh(core_axis_name="c", subcore_axis_name="s")
    n_per_w = _NSUB * _BKCAP

    @functools.partial(
        pl.kernel,
        mesh=mesh,
        out_type=jax.ShapeDtypeStruct((_B + 8, _D), jnp.float32),
        scratch_types=[
            pltpu.VMEM((n_per_w, _D), jnp.float32),
            pltpu.VMEM((n_per_w + _L,), jnp.int32),
            pltpu.SemaphoreType.DMA,
        ],
        compiler_params=pltpu.CompilerParams(use_tc_tiling_on_sc=False),
    )
    def k(staged_hbm, pos_hbm, out_hbm, rows_v, pos_v, sem):
        wid = lax.axis_index("s") * _NC + lax.axis_index("c")
        base = wid * n_per_w
        pltpu.sync_copy(staged_hbm.at[pl.ds(base, n_per_w)], rows_v)
        pltpu.sync_copy(pos_hbm.at[pl.ds(base, n_per_w)],
                        pos_v.at[pl.ds(0, n_per_w)])

        def fire(e, carry):
            p = pos_v[pl.ds(e, _L)][0]
            pltpu.async_copy(rows_v.at[e], out_hbm.at[p], sem)
            return carry

        lax.fori_loop(0, n_per_w, fire, 0)

        def drain(e, carry):
            pltpu.make_async_copy(rows_v.at[0], out_hbm.at[0], sem).wait()
            return carry

        lax.fori_loop(0, n_per_w, drain, 0)

    return k(staged, pos)


_BLK = 2048


def _mlp_body(emb_ref, w1_ref, b1_ref, w2_ref, b2_ref, out_ref):
    emb = emb_ref[...]
    h = jnp.dot(emb, w1_ref[...], preferred_element_type=jnp.float32)
    h = h + b1_ref[...]
    h = 0.5 * h * (1.0 + lax.erf(h * (1.0 / math.sqrt(2.0))))
    o = jnp.dot(h, w2_ref[...], preferred_element_type=jnp.float32)
    out_ref[...] = o + b2_ref[...] + emb


def _mlp(emb, W1, b1, W2, b2):
    grid = (_B // _BLK,)
    return pl.pallas_call(
        _mlp_body,
        grid=grid,
        in_specs=[
            pl.BlockSpec((_BLK, _D), lambda i: (i, 0)),
            pl.BlockSpec((_D, 2 * _D), lambda i: (0, 0)),
            pl.BlockSpec((1, 2 * _D), lambda i: (0, 0)),
            pl.BlockSpec((2 * _D, _D), lambda i: (0, 0)),
            pl.BlockSpec((1, _D), lambda i: (0, 0)),
        ],
        out_specs=pl.BlockSpec((_BLK, _D), lambda i: (i, 0)),
        out_shape=jax.ShapeDtypeStruct((_B, _D), jnp.float32),
    )(emb, W1, b1, W2, b2)


def kernel(table, W1, b1, W2, b2, device_idx, dose_idx):
    dev = device_idx.astype(jnp.int32)
    dose = dose_idx.astype(jnp.int32)
    staged, pos = _sc_scan_select(dev, dose, table.T)
    emb = _sc_scatter(staged, pos)[:_B]
    return _mlp(emb, W1, b1.reshape(1, -1), W2, b2.reshape(1, -1))


# speedup vs baseline: 3.2038x; 3.2038x over previous
import functools
import math

import jax
import jax.numpy as jnp
from jax import lax
from jax.experimental import pallas as pl
from jax.experimental.pallas import tpu as pltpu
from jax.experimental.pallas import tpu_sc as plsc

_NUM_DOSES = 100
_B = 16384
_D = 64
_V = 1000000
_NC = 2
_NS = 16
_NW = _NC * _NS
_L = 16
_NBLK = (_V + 127) // 128
_BPB = _NBLK // _NW
_CAP = 768
_NSUB = 16
_SB = 16
_BKCAP = 80
_CHF = 2048
_SENT = 0x7FFFFFFF
_PSENT = _B


def _sc_scan_select(dev, dose, table_t):
    mesh = plsc.VectorSubcoreMesh(core_axis_name="c", subcore_axis_name="s")

    @functools.partial(
        pl.kernel,
        mesh=mesh,
        out_type=(
            jax.ShapeDtypeStruct((_NW * _NSUB * _BKCAP, _D), jnp.float32),
            jax.ShapeDtypeStruct((_NW * _NSUB * _BKCAP,), jnp.int32),
            jax.ShapeDtypeStruct((_NW * _NSUB * _L,), jnp.int32),
        ),
        scratch_types=[
            pltpu.VMEM((_CHF,), jnp.int32),
            pltpu.VMEM((_CHF,), jnp.int32),
            pltpu.VMEM((_CAP + _L,), jnp.int32),
            pltpu.VMEM((_CAP + _L,), jnp.int32),
            pltpu.VMEM((_NSUB, _BKCAP + _L), jnp.int32),
            pltpu.VMEM((_NSUB, _BKCAP + _L), jnp.int32),
            pltpu.VMEM((_BKCAP + _L,), jnp.int32),
            pltpu.VMEM((_BKCAP + _L,), jnp.int32),
            pltpu.VMEM((64, 1024), jnp.float32),
            pltpu.VMEM((_BKCAP, _D), jnp.float32),
            pltpu.VMEM((_L,), jnp.int32),
        ],
        compiler_params=pltpu.CompilerParams(
            use_tc_tiling_on_sc=True, needs_layout_passes=False),
    )
    def k(dev_hbm, dose_hbm, table_hbm, staged_hbm, pos_hbm, cnt_hbm,
          dv, sv, oidx, opos, bidx, bpos, mrow, ment, buf, rows_s, cnt_v):
        wid = lax.axis_index("s") * _NC + lax.axis_index("c")
        nb = _BPB + ((wid + 1) // _NW) * (_NBLK - _NW * _BPB)
        lo_b = wid * _BPB
        iota = lax.iota(jnp.int32, _L)
        sentv = jnp.full((_L,), _SENT, jnp.int32)
        psentv = jnp.full((_L,), _PSENT, jnp.int32)

        def initb(v, c):
            oidx[pl.ds(v * _L, _L)] = sentv
            return c
        lax.fori_loop(0, (_CAP + _L) // _L, initb, 0)
        for s in range(_NSUB):
            def initp(v, c, s=s):
                bpos[s, pl.ds(v * _L, _L)] = psentv
                bidx[s, pl.ds(v * _L, _L)] = sentv
                return c
            lax.fori_loop(0, (_BKCAP + _L) // _L, initp, 0)

        def chunk(ci, cnt):
            pltpu.sync_copy(dev_hbm.at[pl.ds(ci * _CHF, _CHF)], dv)
            pltpu.sync_copy(dose_hbm.at[pl.ds(ci * _CHF, _CHF)], sv)

            def vec(vi, cnt):
                sl = pl.ds(vi * _L, _L)
                c = dv[sl] * _NUM_DOSES + sv[sl]
                b = lax.shift_right_logical(c, 7)
                m = jnp.logical_and(b >= lo_b, b < lo_b + nb)
                key = jnp.where(m, c, jnp.int32(_SENT))
                p = iota + (ci * _CHF + vi * _L)
                sk, sp = plsc.sort_key_val(key, p)
                oidx[pl.ds(cnt, _L)] = sk
                opos[pl.ds(cnt, _L)] = sp
                return lax.min(cnt + jnp.sum(m.astype(jnp.int32)), _CAP)

            return lax.fori_loop(0, _CHF // _L, vec, cnt)

        cnt = lax.fori_loop(0, _B // _CHF, chunk, 0)
        nvec = lax.shift_right_logical(cnt + _L - 1, 4)

        bcs = []
        for s in range(_NSUB):
            def bucket(v, bc, s=s):
                ob = oidx[pl.ds(v * _L, _L)]
                op = opos[pl.ds(v * _L, _L)]
                sub = lax.shift_right_logical(
                    lax.shift_right_logical(ob, 7) - lo_b, 4)
                m = sub == jnp.full((_L,), s, jnp.int32)
                key = jnp.where(m, ob, jnp.int32(_SENT))
                sk, sp = plsc.sort_key_val(key, jnp.where(m, op, psentv))
                bidx[s, pl.ds(bc, _L)] = sk
                bpos[s, pl.ds(bc, _L)] = sp
                return lax.min(bc + jnp.sum(m.astype(jnp.int32)), _BKCAP)

            bcs.append(lax.fori_loop(0, nvec, bucket, 0))

        for s in range(_NSUB):
            nv_s = lax.shift_right_logical(bcs[s] + _L - 1, 4)
            for h in range(2):
                blkh_lo = lo_b + s * _SB + h * 8

                @pl.when(blkh_lo < lo_b + nb)
                def _(s=s, h=h, blkh_lo=blkh_lo, nv_s=nv_s):
                    off = pl.multiple_of(
                        lax.min(blkh_lo * 128, (_NBLK * 128 + 64) - 1024), 128)
                    pltpu.sync_copy(table_hbm.at[:, pl.ds(off, 1024)], buf)
                    lov = jnp.full((_L,), blkh_lo, jnp.int32)

                    def scan(v, mcnt):
                        ob = bidx[s, pl.ds(v * _L, _L)]
                        b = lax.shift_right_logical(ob, 7)
                        m = jnp.logical_and(b >= lov, b < lov + 8)
                        key = jnp.where(m, ob - off, jnp.int32(_SENT))
                        sk, se = plsc.sort_key_val(key, iota + v * _L)
                        mrow[pl.ds(mcnt, _L)] = sk
                        ment[pl.ds(mcnt, _L)] = se
                        return mcnt + jnp.sum(m.astype(jnp.int32))

                    mcnt = lax.fori_loop(0, nv_s, scan, 0)

                    def sel(e2, carry2):
                        r = mrow[pl.ds(e2, _L)][0]
                        e = ment[pl.ds(e2, _L)][0]
                        rsp = jnp.full((_L,), r, jnp.int32)
                        for g in range(_D // _L):
                            col = plsc.load_gather(buf, [iota + g * _L, rsp])
                            rows_s[e, pl.ds(g * _L, _L)] = col
                        return carry2

                    lax.fori_loop(0, mcnt, sel, 0)

            base = wid * _NSUB * _BKCAP + s * _BKCAP
            pltpu.sync_copy(rows_s, staged_hbm.at[pl.ds(base, _BKCAP)])
            pltpu.sync_copy(bpos.at[s, pl.ds(0, _BKCAP)],
                            pos_hbm.at[pl.ds(base, _BKCAP)])
            cnt_v[...] = jnp.full((_L,), bcs[s], jnp.int32)
            pltpu.sync_copy(
                cnt_v, cnt_hbm.at[pl.ds((wid * _NSUB + s) * _L, _L)])

    return k(dev, dose, table_t)


def _sc_scatter(staged, pos, cnt):
    mesh = plsc.VectorSubcoreMesh(core_axis_name="c", subcore_axis_name="s")
    n_per_w = _NSUB * _BKCAP

    @functools.partial(
        pl.kernel,
        mesh=mesh,
        out_type=jax.ShapeDtypeStruct((_B + 8, _D), jnp.float32),
        scratch_types=[
            pltpu.VMEM((n_per_w, _D), jnp.float32),
            pltpu.VMEM((n_per_w + _L,), jnp.int32),
            pltpu.VMEM((_NSUB * _L,), jnp.int32),
            pltpu.SemaphoreType.DMA,
        ],
        compiler_params=pltpu.CompilerParams(use_tc_tiling_on_sc=False),
    )
    def k(staged_hbm, pos_hbm, cnt_hbm, out_hbm, rows_v, pos_v, cnt_v, sem):
        wid = lax.axis_index("s") * _NC + lax.axis_index("c")
        base = wid * n_per_w
        pltpu.sync_copy(staged_hbm.at[pl.ds(base, n_per_w)], rows_v)
        pltpu.sync_copy(pos_hbm.at[pl.ds(base, n_per_w)],
                        pos_v.at[pl.ds(0, n_per_w)])
        pltpu.sync_copy(cnt_hbm.at[pl.ds(wid * _NSUB * _L, _NSUB * _L)], cnt_v)

        total = 0
        for s in range(_NSUB):
            n_s = cnt_v[pl.ds(s * _L, _L)][0]
            sbase = s * _BKCAP

            def fire(e, carry, sbase=sbase):
                p = pos_v[pl.ds(sbase + e, _L)][0]
                pltpu.async_copy(rows_v.at[sbase + e], out_hbm.at[p], sem)
                return carry

            lax.fori_loop(0, n_s, fire, 0)
            total = total + n_s

        def drain(e, carry):
            pltpu.make_async_copy(rows_v.at[0], out_hbm.at[0], sem).wait()
            return carry

        lax.fori_loop(0, total, drain, 0)

    return k(staged, pos, cnt)


_BLK = 2048


def _mlp_body(emb_ref, w1_ref, b1_ref, w2_ref, b2_ref, out_ref):
    emb = emb_ref[...]
    h = jnp.dot(emb, w1_ref[...], preferred_element_type=jnp.float32)
    h = h + b1_ref[...]
    h = 0.5 * h * (1.0 + lax.erf(h * (1.0 / math.sqrt(2.0))))
    o = jnp.dot(h, w2_ref[...], preferred_element_type=jnp.float32)
    out_ref[...] = o + b2_ref[...] + emb


def _mlp(emb, W1, b1, W2, b2):
    grid = (_B // _BLK,)
    return pl.pallas_call(
        _mlp_body,
        grid=grid,
        in_specs=[
            pl.BlockSpec((_BLK, _D), lambda i: (i, 0)),
            pl.BlockSpec((_D, 2 * _D), lambda i: (0, 0)),
            pl.BlockSpec((1, 2 * _D), lambda i: (0, 0)),
            pl.BlockSpec((2 * _D, _D), lambda i: (0, 0)),
            pl.BlockSpec((1, _D), lambda i: (0, 0)),
        ],
        out_specs=pl.BlockSpec((_BLK, _D), lambda i: (i, 0)),
        out_shape=jax.ShapeDtypeStruct((_B, _D), jnp.float32),
    )(emb, W1, b1, W2, b2)


def kernel(table, W1, b1, W2, b2, device_idx, dose_idx):
    dev = device_idx.astype(jnp.int32)
    dose = dose_idx.astype(jnp.int32)
    staged, pos, cnt = _sc_scan_select(dev, dose, table.T)
    emb = _sc_scatter(staged, pos, cnt)[:_B]
    return _mlp(emb, W1, b1.reshape(1, -1), W2, b2.reshape(1, -1))


# speedup vs baseline: 3.2072x; 1.0010x over previous
import functools
import math

import jax
import jax.numpy as jnp
from jax import lax
from jax.experimental import pallas as pl
from jax.experimental.pallas import tpu as pltpu
from jax.experimental.pallas import tpu_sc as plsc

_NUM_DOSES = 100
_B = 16384
_D = 64
_V = 1000000
_NC = 2
_NS = 16
_NW = _NC * _NS
_L = 16
_NBLK = (_V + 127) // 128
_BPB = _NBLK // _NW
_CAP = 768
_NSUB = 16
_SB = 16
_BKCAP = 80
_CHF = 2048
_SENT = 0x7FFFFFFF
_PSENT = _B


def _sc_scan_select(dev, dose, table_t):
    mesh = plsc.VectorSubcoreMesh(core_axis_name="c", subcore_axis_name="s")

    @functools.partial(
        pl.kernel,
        mesh=mesh,
        out_type=(
            jax.ShapeDtypeStruct((_NW * _NSUB * _BKCAP, _D), jnp.float32),
            jax.ShapeDtypeStruct((_NW * _NSUB * _BKCAP,), jnp.int32),
            jax.ShapeDtypeStruct((_NW * _NSUB * _L,), jnp.int32),
        ),
        scratch_types=[
            pltpu.VMEM((64, 64), jnp.float32),
            pltpu.VMEM((_CHF,), jnp.int32),
            pltpu.VMEM((_CHF,), jnp.int32),
            pltpu.VMEM((_CAP + _L,), jnp.int32),
            pltpu.VMEM((_CAP + _L,), jnp.int32),
            pltpu.VMEM((_NSUB, _BKCAP + _L), jnp.int32),
            pltpu.VMEM((_NSUB, _BKCAP + _L), jnp.int32),
            pltpu.VMEM((_BKCAP + _L,), jnp.int32),
            pltpu.VMEM((_BKCAP + _L,), jnp.int32),
            pltpu.VMEM((64, 1024), jnp.float32),
            pltpu.VMEM((_BKCAP, _D), jnp.float32),
            pltpu.VMEM((_L,), jnp.int32),
        ],
        compiler_params=pltpu.CompilerParams(
            use_tc_tiling_on_sc=True, needs_layout_passes=False),
    )
    def k(dev_hbm, dose_hbm, table_hbm, tail_hbm, staged_hbm, pos_hbm,
          cnt_hbm, tailv, dv, sv, oidx, opos, bidx, bpos, mrow, ment, buf,
          rows_s, cnt_v):
        wid = lax.axis_index("s") * _NC + lax.axis_index("c")
        nb = _BPB + ((wid + 1) // _NW) * (_NBLK - _NW * _BPB)
        lo_b = wid * _BPB
        iota = lax.iota(jnp.int32, _L)
        sentv = jnp.full((_L,), _SENT, jnp.int32)
        psentv = jnp.full((_L,), _PSENT, jnp.int32)

        def initb(v, c):
            oidx[pl.ds(v * _L, _L)] = sentv
            return c
        lax.fori_loop(0, (_CAP + _L) // _L, initb, 0)
        for s in range(_NSUB):
            def initp(v, c, s=s):
                bpos[s, pl.ds(v * _L, _L)] = psentv
                bidx[s, pl.ds(v * _L, _L)] = sentv
                return c
            lax.fori_loop(0, (_BKCAP + _L) // _L, initp, 0)

        def chunk(ci, cnt):
            pltpu.sync_copy(dev_hbm.at[pl.ds(ci * _CHF, _CHF)], dv)
            pltpu.sync_copy(dose_hbm.at[pl.ds(ci * _CHF, _CHF)], sv)

            def vec(vi, cnt):
                sl = pl.ds(vi * _L, _L)
                c = dv[sl] * _NUM_DOSES + sv[sl]
                b = lax.shift_right_logical(c, 7)
                m = jnp.logical_and(b >= lo_b, b < lo_b + nb)
                key = jnp.where(m, c, jnp.int32(_SENT))
                p = iota + (ci * _CHF + vi * _L)
                sk, sp = plsc.sort_key_val(key, p)
                oidx[pl.ds(cnt, _L)] = sk
                opos[pl.ds(cnt, _L)] = sp
                return lax.min(cnt + jnp.sum(m.astype(jnp.int32)), _CAP)

            return lax.fori_loop(0, _CHF // _L, vec, cnt)

        cnt = lax.fori_loop(0, _B // _CHF, chunk, 0)
        nvec = lax.shift_right_logical(cnt + _L - 1, 4)

        bcs = []
        for s in range(_NSUB):
            def bucket(v, bc, s=s):
                ob = oidx[pl.ds(v * _L, _L)]
                op = opos[pl.ds(v * _L, _L)]
                sub = lax.shift_right_logical(
                    lax.shift_right_logical(ob, 7) - lo_b, 4)
                m = sub == jnp.full((_L,), s, jnp.int32)
                key = jnp.where(m, ob, jnp.int32(_SENT))
                sk, sp = plsc.sort_key_val(key, jnp.where(m, op, psentv))
                bidx[s, pl.ds(bc, _L)] = sk
                bpos[s, pl.ds(bc, _L)] = sp
                return lax.min(bc + jnp.sum(m.astype(jnp.int32)), _BKCAP)

            bcs.append(lax.fori_loop(0, nvec, bucket, 0))

        for s in range(_NSUB):
            nv_s = lax.shift_right_logical(bcs[s] + _L - 1, 4)
            for h in range(2):
                blkh_lo = lo_b + s * _SB + h * 8

                @pl.when(blkh_lo < lax.min(lo_b + nb, _NBLK - 1))
                def _(s=s, h=h, blkh_lo=blkh_lo, nv_s=nv_s):
                    off = pl.multiple_of(blkh_lo * 128, 128)
                    pltpu.sync_copy(table_hbm.at[:, pl.ds(off, 1024)], buf)
                    lov = jnp.full((_L,), blkh_lo, jnp.int32)

                    def scan(v, mcnt):
                        ob = bidx[s, pl.ds(v * _L, _L)]
                        b = lax.shift_right_logical(ob, 7)
                        m = jnp.logical_and(b >= lov, b < lov + 8)
                        key = jnp.where(m, ob - off, jnp.int32(_SENT))
                        sk, se = plsc.sort_key_val(key, iota + v * _L)
                        mrow[pl.ds(mcnt, _L)] = sk
                        ment[pl.ds(mcnt, _L)] = se
                        return mcnt + jnp.sum(m.astype(jnp.int32))

                    mcnt = lax.fori_loop(0, nv_s, scan, 0)

                    def sel(e2, carry2):
                        r = mrow[pl.ds(e2, _L)][0]
                        e = ment[pl.ds(e2, _L)][0]
                        rsp = jnp.full((_L,), r, jnp.int32)
                        for g in range(_D // _L):
                            col = plsc.load_gather(buf, [iota + g * _L, rsp])
                            rows_s[e, pl.ds(g * _L, _L)] = col
                        return carry2

                    lax.fori_loop(0, mcnt, sel, 0)

            if s == _NSUB - 1:
                @pl.when(lo_b + nb == _NBLK)
                def _(nv_s=nv_s):
                    pltpu.sync_copy(tail_hbm, tailv)
                    lastv = jnp.full((_L,), _NBLK - 1, jnp.int32)

                    def tscan(v, mcnt):
                        ob = bidx[_NSUB - 1, pl.ds(v * _L, _L)]
                        m = lax.shift_right_logical(ob, 7) == lastv
                        key = jnp.where(m, ob - (_V - 64), jnp.int32(_SENT))
                        sk, se = plsc.sort_key_val(key, iota + v * _L)
                        mrow[pl.ds(mcnt, _L)] = sk
                        ment[pl.ds(mcnt, _L)] = se
                        return mcnt + jnp.sum(m.astype(jnp.int32))

                    tmcnt = lax.fori_loop(0, nv_s, tscan, 0)

                    def tsel(e2, carry2):
                        r = mrow[pl.ds(e2, _L)][0]
                        e = ment[pl.ds(e2, _L)][0]
                        rsp = jnp.full((_L,), r, jnp.int32)
                        for g in range(_D // _L):
                            col = plsc.load_gather(tailv, [iota + g * _L, rsp])
                            rows_s[e, pl.ds(g * _L, _L)] = col
                        return carry2

                    lax.fori_loop(0, tmcnt, tsel, 0)

            base = wid * _NSUB * _BKCAP + s * _BKCAP
            pltpu.sync_copy(rows_s, staged_hbm.at[pl.ds(base, _BKCAP)])
            pltpu.sync_copy(bpos.at[s, pl.ds(0, _BKCAP)],
                            pos_hbm.at[pl.ds(base, _BKCAP)])
            cnt_v[...] = jnp.full((_L,), bcs[s], jnp.int32)
            pltpu.sync_copy(
                cnt_v, cnt_hbm.at[pl.ds((wid * _NSUB + s) * _L, _L)])

    return k(dev, dose, table_t, table_t[:, _V - 64:])


def _sc_scatter(staged, pos, cnt):
    mesh = plsc.VectorSubcoreMesh(core_axis_name="c", subcore_axis_name="s")
    n_per_w = _NSUB * _BKCAP

    @functools.partial(
        pl.kernel,
        mesh=mesh,
        out_type=jax.ShapeDtypeStruct((_B + 8, _D), jnp.float32),
        scratch_types=[
            pltpu.VMEM((n_per_w, _D), jnp.float32),
            pltpu.VMEM((n_per_w + _L,), jnp.int32),
            pltpu.VMEM((_NSUB * _L,), jnp.int32),
            pltpu.SemaphoreType.DMA,
        ],
        compiler_params=pltpu.CompilerParams(use_tc_tiling_on_sc=False),
    )
    def k(staged_hbm, pos_hbm, cnt_hbm, out_hbm, rows_v, pos_v, cnt_v, sem):
        wid = lax.axis_index("s") * _NC + lax.axis_index("c")
        base = wid * n_per_w
        pltpu.sync_copy(staged_hbm.at[pl.ds(base, n_per_w)], rows_v)
        pltpu.sync_copy(pos_hbm.at[pl.ds(base, n_per_w)],
                        pos_v.at[pl.ds(0, n_per_w)])
        pltpu.sync_copy(cnt_hbm.at[pl.ds(wid * _NSUB * _L, _NSUB * _L)], cnt_v)

        total = 0
        for s in range(_NSUB):
            n_s = cnt_v[pl.ds(s * _L, _L)][0]
            sbase = s * _BKCAP

            def fire(e, carry, sbase=sbase):
                p = pos_v[pl.ds(sbase + e, _L)][0]
                pltpu.async_copy(rows_v.at[sbase + e], out_hbm.at[p], sem)
                return carry

            lax.fori_loop(0, n_s, fire, 0)
            total = total + n_s

        def drain(e, carry):
            pltpu.make_async_copy(rows_v.at[0], out_hbm.at[0], sem).wait()
            return carry

        lax.fori_loop(0, total, drain, 0)

    return k(staged, pos, cnt)


_BLK = 2048


def _mlp_body(emb_ref, w1_ref, b1_ref, w2_ref, b2_ref, out_ref):
    emb = emb_ref[...]
    h = jnp.dot(emb, w1_ref[...], preferred_element_type=jnp.float32)
    h = h + b1_ref[...]
    h = 0.5 * h * (1.0 + lax.erf(h * (1.0 / math.sqrt(2.0))))
    o = jnp.dot(h, w2_ref[...], preferred_element_type=jnp.float32)
    out_ref[...] = o + b2_ref[...] + emb


def _mlp(emb, W1, b1, W2, b2):
    grid = (_B // _BLK,)
    return pl.pallas_call(
        _mlp_body,
        grid=grid,
        in_specs=[
            pl.BlockSpec((_BLK, _D), lambda i: (i, 0)),
            pl.BlockSpec((_D, 2 * _D), lambda i: (0, 0)),
            pl.BlockSpec((1, 2 * _D), lambda i: (0, 0)),
            pl.BlockSpec((2 * _D, _D), lambda i: (0, 0)),
            pl.BlockSpec((1, _D), lambda i: (0, 0)),
        ],
        out_specs=pl.BlockSpec((_BLK, _D), lambda i: (i, 0)),
        out_shape=jax.ShapeDtypeStruct((_B, _D), jnp.float32),
    )(emb, W1, b1, W2, b2)


def kernel(table, W1, b1, W2, b2, device_idx, dose_idx):
    dev = device_idx.astype(jnp.int32)
    dose = dose_idx.astype(jnp.int32)
    tt = table.T
    staged, pos, cnt = _sc_scan_select(dev, dose, tt)
    emb = _sc_scatter(staged, pos, cnt)[:_B]
    return _mlp(emb, W1, b1.reshape(1, -1), W2, b2.reshape(1, -1))
